# Initial kernel scaffold; baseline (speedup 1.0000x reference)
import jax
import jax.numpy as jnp
from jax.experimental import pallas as pl


def kernel(x, Y, W_e, W1, b1, W2, b2):
    raise NotImplementedError("write your pallas kernel here")


# speedup vs baseline: 4.0975x; 4.0975x over previous
import functools

import jax
import jax.numpy as jnp
from jax import lax
from jax.experimental import pallas as pl
from jax.experimental.pallas import tpu as pltpu
from jax.experimental.pallas import tpu_sc as plsc

D = 64
DH = 128
EPS = 1e-08
B, N, T = 1024, 8, 200
T_PAD = 208
HALF = 104
SEGS = B * (N + 1)
NW = 32
SEG_W = SEGS // NW
NBUF = 3

@functools.cache
def _emb_sums_kernel():
    mesh = plsc.VectorSubcoreMesh(core_axis_name="c", subcore_axis_name="s")
    return pl.kernel(
        _emb_sums_body,
        mesh=mesh,
        compiler_params=pltpu.CompilerParams(use_tc_tiling_on_sc=False),
        out_type=jax.ShapeDtypeStruct((SEGS, D), jnp.float32),
        scratch_types=[
            pltpu.VMEM((SEG_W * 2, HALF), jnp.int32),
            pltpu.VMEM((NBUF, T_PAD, D), jnp.float32),
            pltpu.VMEM((16, D), jnp.float32),
            pltpu.SemaphoreType.DMA,
            pltpu.SemaphoreType.DMA,
            pltpu.SemaphoreType.DMA,
        ],
    )


def _emb_sums_body(idx_hbm, we_hbm, out_hbm, idx_v, rows_v, hstage, sem0, sem1, sem2):
    sems = (sem0, sem1, sem2)
    wid = lax.axis_index("s") * 2 + lax.axis_index("c")
    seg0 = pl.multiple_of(wid * SEG_W, SEG_W)
    pltpu.sync_copy(idx_hbm.at[pl.ds(seg0 * 2, SEG_W * 2)], idx_v)

    def start_gather(sl, b):
        pltpu.make_async_copy(
            we_hbm.at[idx_v.at[2 * sl]],
            rows_v.at[b, pl.ds(0, HALF)], sems[b]).start()
        pltpu.make_async_copy(
            we_hbm.at[idx_v.at[2 * sl + 1]],
            rows_v.at[b, pl.ds(HALF, HALF)], sems[b]).start()

    def wait_gather(b):
        pltpu.make_async_copy(
            we_hbm.at[pl.ds(0, T_PAD)], rows_v.at[b], sems[b]).wait()

    for b in range(NBUF):
        start_gather(b, b)

    def gbody(g, carry):
        for b in range(NBUF):
            s = g * NBUF + b
            wait_gather(b)
            zero = jnp.zeros((16,), jnp.float32)

            def acc_chunk(c, accs, b=b):
                a0, a1, a2, a3 = accs
                r0 = c * 16
                for tt in range(16):
                    r = r0 + tt
                    a0 = a0 + rows_v[b, r, pl.ds(0, 16)]
                    a1 = a1 + rows_v[b, r, pl.ds(16, 16)]
                    a2 = a2 + rows_v[b, r, pl.ds(32, 16)]
                    a3 = a3 + rows_v[b, r, pl.ds(48, 16)]
                return (a0, a1, a2, a3)

            a0, a1, a2, a3 = lax.fori_loop(
                0, T_PAD // 16, acc_chunk, (zero, zero, zero, zero))
            s16 = lax.rem(s, 16)
            hstage[s16, pl.ds(0, 16)] = a0
            hstage[s16, pl.ds(16, 16)] = a1
            hstage[s16, pl.ds(32, 16)] = a2
            hstage[s16, pl.ds(48, 16)] = a3

            @pl.when(s16 == 15)
            def _(s=s):
                blk = pl.multiple_of(seg0 + s - 15, 16)
                pltpu.sync_copy(hstage, out_hbm.at[pl.ds(blk, 16)])

            @pl.when(s + NBUF < SEG_W)
            def _(s=s, b=b):
                start_gather(s + NBUF, b)
        return carry

    lax.fori_loop(0, SEG_W // NBUF, gbody, 0)


_ERF_T = (7.853861353153693e-5, -8.010193625184903e-4, 5.188327685732524e-3,
          -2.685381193529856e-2, 1.128358514861418e-1, -3.761262582423300e-1,
          1.128379165726710e+0)
_ERFC_P = (2.326819970068386e-2, -1.387039388740657e-1, 3.687424674597105e-1,
           -5.824733027278666e-1, 6.210004621745983e-1, -4.944515323274145e-1,
           3.404879937665872e-1, -2.741127028184656e-1, 5.638259427386472e-1)
_ERFC_R = (-1.047766399936249e+1, 1.297719955372516e+1, -7.495518717768503e+0,
           2.921019019210786e+0, -1.015265279202700e+0, 4.218463358204948e-1,
           -2.820767439740514e-1, 5.641895067754075e-1)


def _poly(xv, coeffs):
    p = jnp.zeros_like(xv)
    for c in coeffs:
        p = p * xv + jnp.float32(c)
    return p


def _erfc(xv):
    ax = jnp.abs(xv)
    small = 1.0 - xv * _poly(xv * xv, _ERF_T)
    z = jnp.exp(-xv * xv)
    q = 1.0 / ax
    yq = q * q
    p = jnp.where(ax < 2.0, _poly(yq, _ERFC_P), _poly(yq, _ERFC_R))
    yv = z * q * p
    yv = jnp.where(xv < 0.0, 2.0 - yv, yv)
    return jnp.where(ax > 1.0, yv, small)


def _score_body(x_ref, y2_ref, sx_ref, sy_ref,
                w1_ref, b1_ref, w2_ref, b2_ref,
                r_ref, ystar_ref, nstar_ref):
    f32 = jnp.float32
    xcnt = jnp.sum((x_ref[...] != 0).astype(f32), axis=1, keepdims=True)
    ycnt = jnp.sum((y2_ref[...] != 0).astype(f32), axis=1, keepdims=True)
    hx = sx_ref[...] / jnp.maximum(xcnt, EPS)
    hy = sy_ref[...] / jnp.maximum(ycnt, EPS)

    hxr = jnp.broadcast_to(hx[:, None, :], (B, N, D)).reshape(B * N, D)
    phi = jnp.concatenate([hxr, hy, hxr * hy, hy - hxr], axis=1)
    pre = jnp.dot(phi, w1_ref[...], preferred_element_type=f32) + b1_ref[...]
    sqrt_half = jnp.float32(0.7071067811865476)
    hid = 0.5 * pre * _erfc(-pre * sqrt_half)
    r = jnp.dot(hid, w2_ref[...], preferred_element_type=f32) + b2_ref[...]
    r_ref[...] = r

    r3 = r.reshape(B, N, 1)
    rmax = jnp.max(r3, axis=1, keepdims=True)
    nid = lax.broadcasted_iota(jnp.int32, (B, N, 1), 1)
    cand = jnp.where(r3 == rmax, nid, N)
    nstar = jnp.min(cand, axis=1)
    nstar_ref[...] = nstar
    onehot = (nid == nstar[:, None, :]).astype(y2_ref.dtype)
    ysel = y2_ref[...].reshape(B, N, T) * onehot
    ystar_ref[...] = jnp.sum(ysel, axis=1)


def kernel(x, Y, W_e, W1, b1, W2, b2):
    x = x.astype(jnp.int32)
    Y = Y.astype(jnp.int32)
    y2 = Y.reshape(B * N, T)
    W_ext = jnp.concatenate([W_e, jnp.zeros((8, D), jnp.float32)], axis=0)
    xp = jnp.pad(x, ((0, 0), (0, T_PAD - T)))
    yp = jnp.pad(y2, ((0, 0), (0, T_PAD - T)))
    idx_all = jnp.concatenate([xp, yp], axis=0)
    idx_all = jnp.where(idx_all == 0, W_e.shape[0], idx_all)
    idx_all = idx_all.reshape(SEGS * 2, HALF)

    sums = _emb_sums_kernel()(idx_all, W_ext)
    sx = sums[:B]
    sy = sums[B:]

    r8, ystar, nstar1 = pl.pallas_call(
        _score_body,
        out_shape=[
            jax.ShapeDtypeStruct((B * N, 1), jnp.float32),
            jax.ShapeDtypeStruct((B, T), jnp.int32),
            jax.ShapeDtypeStruct((B, 1), jnp.int32),
        ],
    )(x, y2, sx, sy, W1, b1.reshape(1, DH), W2, b2.reshape(1, 1))

    return (r8.reshape(B, N), ystar, nstar1.reshape(B))


# speedup vs baseline: 4.1001x; 1.0006x over previous
import functools

import jax
import jax.numpy as jnp
from jax import lax
from jax.experimental import pallas as pl
from jax.experimental.pallas import tpu as pltpu
from jax.experimental.pallas import tpu_sc as plsc

D = 64
DH = 128
EPS = 1e-08
B, N, T = 1024, 8, 200
T_PAD = 208
HALF = 104
SEGS = B * (N + 1)
NW = 32
SEG_W = SEGS // NW
NBUF = 4

@functools.cache
def _emb_sums_kernel():
    mesh = plsc.VectorSubcoreMesh(core_axis_name="c", subcore_axis_name="s")
    return pl.kernel(
        _emb_sums_body,
        mesh=mesh,
        compiler_params=pltpu.CompilerParams(use_tc_tiling_on_sc=False),
        out_type=jax.ShapeDtypeStruct((SEGS, D), jnp.float32),
        scratch_types=[
            pltpu.VMEM((SEG_W * 2, HALF), jnp.int32),
            pltpu.VMEM((NBUF, T_PAD, D), jnp.float32),
            pltpu.VMEM((16, D), jnp.float32),
            pltpu.SemaphoreType.DMA,
            pltpu.SemaphoreType.DMA,
            pltpu.SemaphoreType.DMA,
            pltpu.SemaphoreType.DMA,
        ],
    )


def _emb_sums_body(idx_hbm, we_hbm, out_hbm, idx_v, rows_v, hstage, sem0, sem1, sem2, sem3):
    sems = (sem0, sem1, sem2, sem3)
    wid = lax.axis_index("s") * 2 + lax.axis_index("c")
    seg0 = pl.multiple_of(wid * SEG_W, SEG_W)
    pltpu.sync_copy(idx_hbm.at[pl.ds(seg0 * 2, SEG_W * 2)], idx_v)

    def start_gather(sl, b):
        pltpu.make_async_copy(
            we_hbm.at[idx_v.at[2 * sl]],
            rows_v.at[b, pl.ds(0, HALF)], sems[b]).start()
        pltpu.make_async_copy(
            we_hbm.at[idx_v.at[2 * sl + 1]],
            rows_v.at[b, pl.ds(HALF, HALF)], sems[b]).start()

    def wait_gather(b):
        pltpu.make_async_copy(
            we_hbm.at[pl.ds(0, T_PAD)], rows_v.at[b], sems[b]).wait()

    for b in range(NBUF):
        start_gather(b, b)

    def gbody(g, carry):
        for b in range(NBUF):
            s = g * NBUF + b
            wait_gather(b)
            zero = jnp.zeros((16,), jnp.float32)

            def acc_chunk(c, accs, b=b):
                a0, a1, a2, a3 = accs
                r0 = c * 16
                for tt in range(16):
                    r = r0 + tt
                    a0 = a0 + rows_v[b, r, pl.ds(0, 16)]
                    a1 = a1 + rows_v[b, r, pl.ds(16, 16)]
                    a2 = a2 + rows_v[b, r, pl.ds(32, 16)]
                    a3 = a3 + rows_v[b, r, pl.ds(48, 16)]
                return (a0, a1, a2, a3)

            a0, a1, a2, a3 = lax.fori_loop(
                0, T_PAD // 16, acc_chunk, (zero, zero, zero, zero))
            s16 = lax.rem(s, 16)
            hstage[s16, pl.ds(0, 16)] = a0
            hstage[s16, pl.ds(16, 16)] = a1
            hstage[s16, pl.ds(32, 16)] = a2
            hstage[s16, pl.ds(48, 16)] = a3

            @pl.when(s16 == 15)
            def _(s=s):
                blk = pl.multiple_of(seg0 + s - 15, 16)
                pltpu.sync_copy(hstage, out_hbm.at[pl.ds(blk, 16)])

            @pl.when(s + NBUF < SEG_W)
            def _(s=s, b=b):
                start_gather(s + NBUF, b)
        return carry

    lax.fori_loop(0, SEG_W // NBUF, gbody, 0)


_ERF_T = (7.853861353153693e-5, -8.010193625184903e-4, 5.188327685732524e-3,
          -2.685381193529856e-2, 1.128358514861418e-1, -3.761262582423300e-1,
          1.128379165726710e+0)
_ERFC_P = (2.326819970068386e-2, -1.387039388740657e-1, 3.687424674597105e-1,
           -5.824733027278666e-1, 6.210004621745983e-1, -4.944515323274145e-1,
           3.404879937665872e-1, -2.741127028184656e-1, 5.638259427386472e-1)
_ERFC_R = (-1.047766399936249e+1, 1.297719955372516e+1, -7.495518717768503e+0,
           2.921019019210786e+0, -1.015265279202700e+0, 4.218463358204948e-1,
           -2.820767439740514e-1, 5.641895067754075e-1)


def _poly(xv, coeffs):
    p = jnp.zeros_like(xv)
    for c in coeffs:
        p = p * xv + jnp.float32(c)
    return p


def _erfc(xv):
    ax = jnp.abs(xv)
    small = 1.0 - xv * _poly(xv * xv, _ERF_T)
    z = jnp.exp(-xv * xv)
    q = 1.0 / ax
    yq = q * q
    p = jnp.where(ax < 2.0, _poly(yq, _ERFC_P), _poly(yq, _ERFC_R))
    yv = z * q * p
    yv = jnp.where(xv < 0.0, 2.0 - yv, yv)
    return jnp.where(ax > 1.0, yv, small)


def _score_body(x_ref, y2_ref, sx_ref, sy_ref,
                w1_ref, b1_ref, w2_ref, b2_ref,
                r_ref, ystar_ref, nstar_ref):
    f32 = jnp.float32
    xcnt = jnp.sum((x_ref[...] != 0).astype(f32), axis=1, keepdims=True)
    ycnt = jnp.sum((y2_ref[...] != 0).astype(f32), axis=1, keepdims=True)
    hx = sx_ref[...] / jnp.maximum(xcnt, EPS)
    hy = sy_ref[...] / jnp.maximum(ycnt, EPS)

    hxr = jnp.broadcast_to(hx[:, None, :], (B, N, D)).reshape(B * N, D)
    phi = jnp.concatenate([hxr, hy, hxr * hy, hy - hxr], axis=1)
    pre = jnp.dot(phi, w1_ref[...], preferred_element_type=f32) + b1_ref[...]
    sqrt_half = jnp.float32(0.7071067811865476)
    hid = 0.5 * pre * _erfc(-pre * sqrt_half)
    r = jnp.dot(hid, w2_ref[...], preferred_element_type=f32) + b2_ref[...]
    r_ref[...] = r

    r3 = r.reshape(B, N, 1)
    rmax = jnp.max(r3, axis=1, keepdims=True)
    nid = lax.broadcasted_iota(jnp.int32, (B, N, 1), 1)
    cand = jnp.where(r3 == rmax, nid, N)
    nstar = jnp.min(cand, axis=1)
    nstar_ref[...] = nstar
    onehot = (nid == nstar[:, None, :]).astype(y2_ref.dtype)
    ysel = y2_ref[...].reshape(B, N, T) * onehot
    ystar_ref[...] = jnp.sum(ysel, axis=1)


def kernel(x, Y, W_e, W1, b1, W2, b2):
    x = x.astype(jnp.int32)
    Y = Y.astype(jnp.int32)
    y2 = Y.reshape(B * N, T)
    W_ext = jnp.concatenate([W_e, jnp.zeros((8, D), jnp.float32)], axis=0)
    xp = jnp.pad(x, ((0, 0), (0, T_PAD - T)))
    yp = jnp.pad(y2, ((0, 0), (0, T_PAD - T)))
    idx_all = jnp.concatenate([xp, yp], axis=0)
    idx_all = jnp.where(idx_all == 0, W_e.shape[0], idx_all)
    idx_all = idx_all.reshape(SEGS * 2, HALF)

    sums = _emb_sums_kernel()(idx_all, W_ext)
    sx = sums[:B]
    sy = sums[B:]

    r8, ystar, nstar1 = pl.pallas_call(
        _score_body,
        out_shape=[
            jax.ShapeDtypeStruct((B * N, 1), jnp.float32),
            jax.ShapeDtypeStruct((B, T), jnp.int32),
            jax.ShapeDtypeStruct((B, 1), jnp.int32),
        ],
    )(x, y2, sx, sy, W1, b1.reshape(1, DH), W2, b2.reshape(1, 1))

    return (r8.reshape(B, N), ystar, nstar1.reshape(B))
